# Initial kernel scaffold; baseline (speedup 1.0000x reference)
import jax
import jax.numpy as jnp
from jax.experimental import pallas as pl


def kernel(x, edge_index, batch, params, fc_W, fc_b):
    raise NotImplementedError("write your pallas kernel here")


# speedup vs baseline: 6.4009x; 6.4009x over previous
import functools

import jax
import jax.numpy as jnp
from jax import lax
from jax.experimental import pallas as pl
from jax.experimental.pallas import tpu as pltpu
from jax.experimental.pallas import tpu_sc as plsc

N = 10000
E = 320000
F = 128
NGRAPH = 64

NTILES = 32
EDGES_PER_TILE = E // NTILES
CHUNK = 80
NCHUNKS = EDGES_PER_TILE // CHUNK
N_PAD = 10240
ROWS_PER_TILE = N_PAD // 16


def _agg_body(h_hbm, src_hbm, dst_hbm, zero_hbm, out_hbm,
              srcv, dstv, rowsv, agg_sh, sem):
    c = lax.axis_index("c")
    s = lax.axis_index("s")
    wid = c * 16 + s
    pltpu.sync_copy(src_hbm.at[wid], srcv)
    pltpu.sync_copy(dst_hbm.at[wid], dstv)
    r0 = s * ROWS_PER_TILE
    pltpu.sync_copy(zero_hbm, agg_sh.at[pl.ds(r0, ROWS_PER_TILE)])
    plsc.subcore_barrier()

    def chunk(j, carry):
        pltpu.async_copy(h_hbm.at[srcv.at[j]], rowsv, sem).wait()
        pltpu.sync_copy(rowsv, agg_sh.at[dstv.at[j]], add=True)
        return carry

    lax.fori_loop(0, NCHUNKS, chunk, 0)
    plsc.subcore_barrier()
    pltpu.sync_copy(agg_sh.at[pl.ds(r0, ROWS_PER_TILE)], out_hbm.at[c].at[pl.ds(r0, ROWS_PER_TILE)])


_agg_call = pl.kernel(
    _agg_body,
    out_type=jax.ShapeDtypeStruct((2, N_PAD, F), jnp.float32),
    mesh=plsc.VectorSubcoreMesh(core_axis_name="c", subcore_axis_name="s"),
    scratch_types=[
        pltpu.VMEM((NCHUNKS, CHUNK), jnp.int32),
        pltpu.VMEM((NCHUNKS, CHUNK), jnp.int32),
        pltpu.VMEM((CHUNK, F), jnp.float32),
        pltpu.VMEM_SHARED((N_PAD, F), jnp.float32),
        pltpu.SemaphoreType.DMA,
    ],
)


def _mlp_body(a0, a1, h, w1, b1, w2, sc, sh, o):
    h2 = a0[...] + a1[...] + h[...]
    z = jnp.dot(h2, w1[...], preferred_element_type=jnp.float32) + b1[...]
    z = jnp.maximum(z, 0.0)
    z = jnp.dot(z, w2[...], preferred_element_type=jnp.float32)
    o[...] = jnp.maximum(z * sc[...] + sh[...], 0.0)


_MLP_ROWS = 1000


def _mlp_call(a0, a1, h, w1, b1, w2, sc, sh):
    row_spec = pl.BlockSpec((_MLP_ROWS, F), lambda i: (i, 0))
    full = pl.BlockSpec((F, F), lambda i: (0, 0))
    vec = pl.BlockSpec((1, F), lambda i: (0, 0))
    return pl.pallas_call(
        _mlp_body,
        grid=(N // _MLP_ROWS,),
        in_specs=[row_spec, row_spec, row_spec, full, vec, full, vec, vec],
        out_specs=row_spec,
        out_shape=jax.ShapeDtypeStruct((N, F), jnp.float32),
    )(a0, a1, h, w1, b1, w2, sc, sh)


def _pool_body(h, batch, fcw, fcb, o):
    ids = batch[...]
    seg = lax.broadcasted_iota(jnp.int32, (NGRAPH, N), 0)
    onehot = (ids == seg).astype(jnp.float32)
    pooled = jnp.dot(onehot, h[...], preferred_element_type=jnp.float32)
    counts = jnp.sum(onehot, axis=1, keepdims=True)
    pooled = pooled / jnp.maximum(counts, 1.0)
    logits = jnp.dot(pooled, fcw[...], preferred_element_type=jnp.float32) + fcb[...]
    mx = jnp.max(logits, axis=1, keepdims=True)
    lse = jnp.log(jnp.sum(jnp.exp(logits - mx), axis=1, keepdims=True))
    o[...] = logits - mx - lse


def _pool_call(h, batch2d, fcw, fcb):
    nclass = fcw.shape[1]
    return pl.pallas_call(
        _pool_body,
        out_shape=jax.ShapeDtypeStruct((NGRAPH, nclass), jnp.float32),
    )(h, batch2d, fcw, fcb)


@jax.jit
def _forward(x, edge_index, batch, params, fc_W, fc_b):
    src3 = edge_index[0].reshape(NTILES, NCHUNKS, CHUNK)
    dst3 = edge_index[1].reshape(NTILES, NCHUNKS, CHUNK)
    zeros = jnp.zeros((ROWS_PER_TILE, F), jnp.float32)
    batch2d = batch.reshape(1, N)
    h = x
    for (W1, b1, W2, b2, g, be, m, v) in params:
        scale = g / jnp.sqrt(v + 1e-5)
        shift = (b2 - m) * scale + be
        parts = _agg_call(h, src3, dst3, zeros)
        h = _mlp_call(parts[0, :N], parts[1, :N], h,
                      W1, b1.reshape(1, F), W2,
                      scale.reshape(1, F), shift.reshape(1, F))
    return _pool_call(h, batch2d, fc_W, fc_b.reshape(1, -1))


def kernel(x, edge_index, batch, params, fc_W, fc_b):
    return _forward(x, edge_index, batch, params, fc_W, fc_b)
